# baseline (speedup 1.0000x reference)
import functools

import jax
import jax.numpy as jnp
from jax import lax
from jax.experimental import pallas as pl
from jax.experimental.pallas import tpu as pltpu
from jax.experimental.pallas import tpu_sc as plsc

_N = 10000
_NP = 10240
_E = 320000
_ITERS = 5
_NT = 16
_EPT = _E // _NT
_NPT = _NP // _NT
_L = 16

_MLP_BLK = 2048
_D = 128
_H = 256


def _mlp_block(x_ref, wi_ref, bi_ref, w0_ref, b0_ref, w1_ref, b1_ref,
               wo_ref, bo_ref, out_ref):
    x = x_ref[...]
    h = jnp.dot(x, wi_ref[...], preferred_element_type=jnp.float32) + bi_ref[...]
    h = jnp.dot(h, w0_ref[...], preferred_element_type=jnp.float32) + b0_ref[...]
    h = jnp.dot(h, w1_ref[...], preferred_element_type=jnp.float32) + b1_ref[...]
    pre = jnp.dot(h, wo_ref[...], preferred_element_type=jnp.float32) + bo_ref[...]
    pre = jnp.maximum(pre, 0.0)
    out_ref[...] = pre[:, 0:1] - pre[:, 1:2]


def _dzlp(Xp, W_in, b_in, W_h0, b_h0, W_h1, b_h1, W_out, b_out):
    grid = _NP // _MLP_BLK
    out = pl.pallas_call(
        _mlp_block,
        grid=(grid,),
        in_specs=[
            pl.BlockSpec((_MLP_BLK, _D), lambda i: (i, 0)),
            pl.BlockSpec((_D, _H), lambda i: (0, 0)),
            pl.BlockSpec((1, _H), lambda i: (0, 0)),
            pl.BlockSpec((_H, _H), lambda i: (0, 0)),
            pl.BlockSpec((1, _H), lambda i: (0, 0)),
            pl.BlockSpec((_H, _H), lambda i: (0, 0)),
            pl.BlockSpec((1, _H), lambda i: (0, 0)),
            pl.BlockSpec((_H, 2), lambda i: (0, 0)),
            pl.BlockSpec((1, 2), lambda i: (0, 0)),
        ],
        out_specs=pl.BlockSpec((_MLP_BLK, 1), lambda i: (i, 0)),
        out_shape=jax.ShapeDtypeStruct((_NP, 1), jnp.float32),
    )(Xp, W_in, b_in.reshape(1, _H), W_h0, b_h0.reshape(1, _H),
      W_h1, b_h1.reshape(1, _H), W_out, b_out.reshape(1, 2))
    return out.reshape(_NP)


_LOG1P_COEFS = (
    0.9999999991556814, -0.49999979491391067, 0.33332497543518685,
    -0.24986496459309737, 0.1988582028428498, -0.16087623006059176,
    0.12377995859002032, -0.08188040908802938, 0.041006573548052316,
    -0.013187826487134938, 0.0019866965759555958,
)


def _sp(x):
    mx = jnp.maximum(x, 0.0)
    mn = jnp.minimum(x, 0.0)
    z = jnp.exp(mn - mx)
    acc = jnp.float32(_LOG1P_COEFS[-1])
    for c in _LOG1P_COEFS[-2::-1]:
        acc = acc * z + jnp.float32(c)
    return mx + z * acc


def _bp_body(ep_hbm, dz_hbm, hb_hbm, out0_hbm, out1_hbm,
             epv, dm, dlp, pa, zs, hv, red, ls, o0, o1, sa, sl):
    c = lax.axis_index("c")
    s = lax.axis_index("s")
    ebase = s * _EPT
    nbase = s * _NPT

    pltpu.sync_copy(ep_hbm.at[pl.ds(ebase, _EPT)], epv)
    pltpu.sync_copy(dz_hbm, dlp)
    pltpu.sync_copy(dz_hbm.at[pl.ds(nbase, _NPT)], zs)
    pltpu.sync_copy(hb_hbm, hv)
    a0 = hv[0]
    a1 = hv[1]
    cc = hv[2]
    zeros = jnp.zeros((_L,), jnp.float32)

    @plsc.parallel_loop(0, _EPT, step=_L, unroll=8)
    def _zinit(b):
        dm[pl.ds(b, _L)] = zeros

    for it in range(_ITERS):
        @plsc.parallel_loop(0, _NP, step=_L, unroll=8)
        def _zagg(b):
            pa[pl.ds(b, _L)] = zeros

        @plsc.parallel_loop(0, _EPT, step=_L, unroll=8)
        def _ebody(b):
            p = epv[pl.ds(b, _L)]
            sc = jnp.bitwise_and(p, 16383)
            nb = lax.shift_right_logical(p, 14)
            g = plsc.load_gather(dlp, [nb])
            d = g - dm[pl.ds(b, _L)]
            dn = cc + _sp(d + a0) - _sp(d + a1)
            dm[pl.ds(b, _L)] = dn
            plsc.addupdate_scatter(pa, [sc], dn)

        pltpu.sync_copy(pa, sa.at[s])
        plsc.subcore_barrier()
        pltpu.sync_copy(sa.at[:, pl.ds(nbase, _NPT)], red)

        @plsc.parallel_loop(0, _NPT, step=_L, unroll=2)
        def _combine(o):
            a = red[0, pl.ds(o, _L)]
            for k in range(1, _NT):
                a = a + red[k, pl.ds(o, _L)]
            ls[pl.ds(o, _L)] = zs[pl.ds(o, _L)] + a

        if it != _ITERS - 1:
            pltpu.sync_copy(ls, sl.at[pl.ds(nbase, _NPT)])
            plsc.subcore_barrier()
            pltpu.sync_copy(sl, dlp)

    @plsc.parallel_loop(0, _NPT, step=_L, unroll=2)
    def _norm(o):
        a = ls[pl.ds(o, _L)]
        o0[pl.ds(o, _L)] = -_sp(-a)
        o1[pl.ds(o, _L)] = -_sp(a)

    @pl.when(c == 0)
    def _():
        pltpu.sync_copy(o0, out0_hbm.at[pl.ds(nbase, _NPT)])
        pltpu.sync_copy(o1, out1_hbm.at[pl.ds(nbase, _NPT)])


@functools.partial(
    pl.kernel,
    out_type=(
        jax.ShapeDtypeStruct((_NP,), jnp.float32),
        jax.ShapeDtypeStruct((_NP,), jnp.float32),
    ),
    mesh=plsc.VectorSubcoreMesh(core_axis_name="c", subcore_axis_name="s"),
    compiler_params=pltpu.CompilerParams(needs_layout_passes=False),
    scratch_types=[
        pltpu.VMEM((_EPT,), jnp.int32),
        pltpu.VMEM((_EPT,), jnp.float32),
        pltpu.VMEM((_NP,), jnp.float32),
        pltpu.VMEM((_NP,), jnp.float32),
        pltpu.VMEM((_NPT,), jnp.float32),
        pltpu.VMEM((4, _L), jnp.float32),
        pltpu.VMEM((_NT, _NPT), jnp.float32),
        pltpu.VMEM((_NPT,), jnp.float32),
        pltpu.VMEM((_NPT,), jnp.float32),
        pltpu.VMEM((_NPT,), jnp.float32),
        pltpu.VMEM_SHARED((_NT, _NP), jnp.float32),
        pltpu.VMEM_SHARED((_NP,), jnp.float32),
    ],
)
def _bp_kernel(ep, dz, hb, out0, out1, *scratch):
    _bp_body(ep, dz, hb, out0, out1, *scratch)


def kernel(edges, X, W_in, b_in, W_h0, b_h0, W_h1, b_h1, W_out, b_out, H):
    Xp = jnp.pad(X, ((0, _NP - _N), (0, 0)))
    dz = _dzlp(Xp, W_in, b_in, W_h0, b_h0, W_h1, b_h1, W_out, b_out)
    src = edges[:, 0].astype(jnp.int32)
    nbr = edges[:, 1].astype(jnp.int32)
    ep = jnp.bitwise_or(src, jnp.left_shift(nbr, 14))
    Hf = H.astype(jnp.float32)
    hrow = jnp.stack([Hf[0, 0] - Hf[0, 1], Hf[1, 0] - Hf[1, 1],
                      Hf[0, 1] - Hf[1, 1], jnp.float32(0.0)])
    hb = jnp.broadcast_to(hrow.reshape(4, 1), (4, _L))
    out0, out1 = _bp_kernel(ep, dz, hb)
    return jnp.stack([out0[:_N], out1[:_N]], axis=1)


# speedup vs baseline: 1.0140x; 1.0140x over previous
import functools

import jax
import jax.numpy as jnp
from jax import lax
from jax.experimental import pallas as pl
from jax.experimental.pallas import tpu as pltpu
from jax.experimental.pallas import tpu_sc as plsc

_N = 10000
_NP = 10240
_E = 320000
_ITERS = 5
_NT = 16
_EPT = _E // _NT
_NPT = _NP // _NT
_L = 16

_MLP_BLK = 1000
_D = 128
_H = 256


def _mlp_block(x_ref, wi_ref, bi_ref, w0_ref, b0_ref, w1_ref, b1_ref,
               wo_ref, bo_ref, out_ref):
    x = x_ref[...]
    h = jnp.dot(x, wi_ref[...], preferred_element_type=jnp.float32) + bi_ref[...]
    h = jnp.dot(h, w0_ref[...], preferred_element_type=jnp.float32) + b0_ref[...]
    h = jnp.dot(h, w1_ref[...], preferred_element_type=jnp.float32) + b1_ref[...]
    pre = jnp.dot(h, wo_ref[...], preferred_element_type=jnp.float32) + bo_ref[...]
    pre = jnp.maximum(pre, 0.0)
    out_ref[...] = pre[:, 0:1] - pre[:, 1:2]


def _dzlp(Xp, W_in, b_in, W_h0, b_h0, W_h1, b_h1, W_out, b_out):
    grid = _N // _MLP_BLK
    out = pl.pallas_call(
        _mlp_block,
        grid=(grid,),
        in_specs=[
            pl.BlockSpec((_MLP_BLK, _D), lambda i: (i, 0)),
            pl.BlockSpec((_D, _H), lambda i: (0, 0)),
            pl.BlockSpec((1, _H), lambda i: (0, 0)),
            pl.BlockSpec((_H, _H), lambda i: (0, 0)),
            pl.BlockSpec((1, _H), lambda i: (0, 0)),
            pl.BlockSpec((_H, _H), lambda i: (0, 0)),
            pl.BlockSpec((1, _H), lambda i: (0, 0)),
            pl.BlockSpec((_H, 2), lambda i: (0, 0)),
            pl.BlockSpec((1, 2), lambda i: (0, 0)),
        ],
        out_specs=pl.BlockSpec((_MLP_BLK, 1), lambda i: (i, 0)),
        out_shape=jax.ShapeDtypeStruct((_N, 1), jnp.float32),
    )(Xp, W_in, b_in.reshape(1, _H), W_h0, b_h0.reshape(1, _H),
      W_h1, b_h1.reshape(1, _H), W_out, b_out.reshape(1, 2))
    return out.reshape(_N)


_LOG1P_COEFS = (
    0.9999999991556814, -0.49999979491391067, 0.33332497543518685,
    -0.24986496459309737, 0.1988582028428498, -0.16087623006059176,
    0.12377995859002032, -0.08188040908802938, 0.041006573548052316,
    -0.013187826487134938, 0.0019866965759555958,
)


def _sp(x):
    mx = jnp.maximum(x, 0.0)
    mn = jnp.minimum(x, 0.0)
    z = jnp.exp(mn - mx)
    acc = jnp.float32(_LOG1P_COEFS[-1])
    for c in _LOG1P_COEFS[-2::-1]:
        acc = acc * z + jnp.float32(c)
    return mx + z * acc


def _bp_body(ep_hbm, dz_hbm, hb_hbm, out0_hbm, out1_hbm,
             epv, dm, dlp, pa, zs, hv, red, ls, o0, o1, sa, sl):
    c = lax.axis_index("c")
    s = lax.axis_index("s")
    ebase = s * _EPT
    nbase = s * _NPT

    pltpu.sync_copy(ep_hbm.at[pl.ds(ebase, _EPT)], epv)
    pltpu.sync_copy(dz_hbm, dlp)
    pltpu.sync_copy(dz_hbm.at[pl.ds(nbase, _NPT)], zs)
    pltpu.sync_copy(hb_hbm, hv)
    a0 = hv[0]
    a1 = hv[1]
    cc = hv[2]
    zeros = jnp.zeros((_L,), jnp.float32)

    for it in range(_ITERS):
        @plsc.parallel_loop(0, _NP, step=_L, unroll=8)
        def _zagg(b):
            pa[pl.ds(b, _L)] = zeros

        @plsc.parallel_loop(0, _EPT, step=_L, unroll=4)
        def _ebody(b, it=it):
            p = epv[pl.ds(b, _L)]
            sc = jnp.bitwise_and(p, 16383)
            nb = lax.shift_right_logical(p, 14)
            g = plsc.load_gather(dlp, [nb])
            d = g if it == 0 else g - dm[pl.ds(b, _L)]
            dn = cc + _sp(d + a0) - _sp(d + a1)
            dm[pl.ds(b, _L)] = dn
            plsc.addupdate_scatter(pa, [sc], dn)

        pltpu.sync_copy(pa, sa.at[s])
        plsc.subcore_barrier()
        pltpu.sync_copy(sa.at[:, pl.ds(nbase, _NPT)], red)

        @plsc.parallel_loop(0, _NPT, step=_L, unroll=2)
        def _combine(o):
            a = red[0, pl.ds(o, _L)]
            for k in range(1, _NT):
                a = a + red[k, pl.ds(o, _L)]
            ls[pl.ds(o, _L)] = zs[pl.ds(o, _L)] + a

        if it != _ITERS - 1:
            pltpu.sync_copy(ls, sl.at[pl.ds(nbase, _NPT)])
            plsc.subcore_barrier()
            pltpu.sync_copy(sl, dlp)

    @plsc.parallel_loop(0, _NPT, step=_L, unroll=2)
    def _norm(o):
        a = ls[pl.ds(o, _L)]
        o0[pl.ds(o, _L)] = -_sp(-a)
        o1[pl.ds(o, _L)] = -_sp(a)

    @pl.when(c == 0)
    def _():
        pltpu.sync_copy(o0, out0_hbm.at[pl.ds(nbase, _NPT)])
        pltpu.sync_copy(o1, out1_hbm.at[pl.ds(nbase, _NPT)])


@functools.partial(
    pl.kernel,
    out_type=(
        jax.ShapeDtypeStruct((_NP,), jnp.float32),
        jax.ShapeDtypeStruct((_NP,), jnp.float32),
    ),
    mesh=plsc.VectorSubcoreMesh(core_axis_name="c", subcore_axis_name="s"),
    compiler_params=pltpu.CompilerParams(needs_layout_passes=False),
    scratch_types=[
        pltpu.VMEM((_EPT,), jnp.int32),
        pltpu.VMEM((_EPT,), jnp.float32),
        pltpu.VMEM((_NP,), jnp.float32),
        pltpu.VMEM((_NP,), jnp.float32),
        pltpu.VMEM((_NPT,), jnp.float32),
        pltpu.VMEM((4, _L), jnp.float32),
        pltpu.VMEM((_NT, _NPT), jnp.float32),
        pltpu.VMEM((_NPT,), jnp.float32),
        pltpu.VMEM((_NPT,), jnp.float32),
        pltpu.VMEM((_NPT,), jnp.float32),
        pltpu.VMEM_SHARED((_NT, _NP), jnp.float32),
        pltpu.VMEM_SHARED((_NP,), jnp.float32),
    ],
)
def _bp_kernel(ep, dz, hb, out0, out1, *scratch):
    _bp_body(ep, dz, hb, out0, out1, *scratch)


def kernel(edges, X, W_in, b_in, W_h0, b_h0, W_h1, b_h1, W_out, b_out, H):
    dz = jnp.pad(_dzlp(X, W_in, b_in, W_h0, b_h0, W_h1, b_h1, W_out, b_out),
                 (0, _NP - _N))
    src = edges[:, 0].astype(jnp.int32)
    nbr = edges[:, 1].astype(jnp.int32)
    ep = jnp.bitwise_or(src, jnp.left_shift(nbr, 14))
    Hf = H.astype(jnp.float32)
    hrow = jnp.stack([Hf[0, 0] - Hf[0, 1], Hf[1, 0] - Hf[1, 1],
                      Hf[0, 1] - Hf[1, 1], jnp.float32(0.0)])
    hb = jnp.broadcast_to(hrow.reshape(4, 1), (4, _L))
    out0, out1 = _bp_kernel(ep, dz, hb)
    return jnp.stack([out0[:_N], out1[:_N]], axis=1)
